# Initial kernel scaffold; baseline (speedup 1.0000x reference)
import jax
import jax.numpy as jnp
from jax.experimental import pallas as pl


def kernel(xyz, points, feature_camera, affine_alpha, affine_beta, conv_w, conv_b, bn_gamma, bn_beta):
    raise NotImplementedError("write your pallas kernel here")


# speedup vs baseline: 4.5667x; 4.5667x over previous
import functools

import jax
import jax.numpy as jnp
from jax import lax
from jax.experimental import pallas as pl
from jax.experimental.pallas import tpu as pltpu
from jax.experimental.pallas import tpu_sc as plsc


def _fps_body(xyzt_ref, idx_ref):
    x3 = xyzt_ref[0]
    n = x3.shape[1]
    s = idx_ref.shape[2]
    iota_n = lax.broadcasted_iota(jnp.int32, (1, n), 1)
    iota_s = lax.broadcasted_iota(jnp.int32, (1, s), 1)

    def body(i, carry):
        dists, far, idxv = carry
        idxv = jnp.where(iota_s == i, far, idxv)
        oh = iota_n == far
        c = jnp.sum(jnp.where(oh, x3, 0.0), axis=1, keepdims=True)
        d = jnp.sum((x3 - c) ** 2, axis=0, keepdims=True)
        dists = jnp.minimum(dists, d)
        m = jnp.max(dists, axis=1, keepdims=True)
        far = jnp.min(jnp.where(dists == m, iota_n, n), axis=1, keepdims=True)
        return dists, far, idxv

    dists0 = jnp.full((1, n), 1e10, jnp.float32)
    far0 = jnp.zeros((1, 1), jnp.int32)
    idxv0 = jnp.zeros((1, s), jnp.int32)
    _, _, idxv = lax.fori_loop(0, s, body, (dists0, far0, idxv0))
    idx_ref[0] = idxv + pl.program_id(0) * n


def _run_fps(xyzt, B, N, S):
    return pl.pallas_call(
        _fps_body,
        grid=(B,),
        in_specs=[pl.BlockSpec((1, 3, N), lambda b: (b, 0, 0))],
        out_specs=pl.BlockSpec((1, 1, S), lambda b: (b, 0, 0)),
        out_shape=jax.ShapeDtypeStruct((B, 1, S), jnp.int32),
    )(xyzt)


def _sc_gather(table, idx, chunk=128):
    rows, depth = idx.shape[0], table.shape[1]
    info = plsc.get_sparse_core_info()
    ncores = info.num_cores
    nworkers = ncores * info.num_subcores
    per_w = rows // nworkers
    n_chunks = per_w // chunk
    mesh = plsc.VectorSubcoreMesh(core_axis_name="c", subcore_axis_name="s")

    @functools.partial(
        pl.kernel,
        mesh=mesh,
        out_type=jax.ShapeDtypeStruct((rows, depth), jnp.float32),
        scratch_types=[
            pltpu.VMEM((chunk,), jnp.int32),
            pltpu.VMEM((chunk, depth), jnp.float32),
            pltpu.SemaphoreType.DMA,
        ],
    )
    def gk(table_hbm, idx_hbm, out_hbm, idx_v, rows_v, sem):
        wid = lax.axis_index("s") * ncores + lax.axis_index("c")
        base = wid * per_w

        def body(i, carry):
            off = base + i * chunk
            pltpu.sync_copy(idx_hbm.at[pl.ds(off, chunk)], idx_v)
            pltpu.async_copy(table_hbm.at[idx_v], rows_v, sem).wait()
            pltpu.sync_copy(rows_v, out_hbm.at[pl.ds(off, chunk)])
            return carry

        lax.fori_loop(0, n_chunks, body, 0)

    return gk(table, idx)


def _knn_body(q_ref, kt_ref, idx_ref, *, N, K, TS):
    q = q_ref[0]
    kt = kt_ref[0]
    qk = lax.dot_general(q, kt, (((1,), (0,)), ((), ())),
                         preferred_element_type=jnp.float32)
    q2 = jnp.sum(q * q, axis=1, keepdims=True)
    k2 = jnp.sum(kt * kt, axis=0, keepdims=True)
    dmat = q2 - 2.0 * qk + k2
    iota = lax.broadcasted_iota(jnp.int32, (TS, N), 1)
    iota_k = lax.broadcasted_iota(jnp.int32, (TS, K), 1)
    acc = jnp.zeros((TS, K), jnp.int32)
    for k in range(K):
        m = jnp.min(dmat, axis=1, keepdims=True)
        a = jnp.min(jnp.where(dmat == m, iota, N), axis=1, keepdims=True)
        acc = jnp.where(iota_k == k, a, acc)
        dmat = jnp.where(iota == a, 1e30, dmat)
    idx_ref[0] = acc + pl.program_id(0) * N


def _run_knn(new_xyz, xyzt, B, N, S, K, TS=128):
    return pl.pallas_call(
        functools.partial(_knn_body, N=N, K=K, TS=TS),
        grid=(B, S // TS),
        in_specs=[
            pl.BlockSpec((1, TS, 3), lambda b, j: (b, j, 0)),
            pl.BlockSpec((1, 3, N), lambda b, j: (b, 0, 0)),
        ],
        out_specs=pl.BlockSpec((1, TS, K), lambda b, j: (b, j, 0)),
        out_shape=jax.ShapeDtypeStruct((B, S, K), jnp.int32),
    )(new_xyz, xyzt)


def _stat_body(g_ref, a_ref, s1_ref, s2_ref):
    g = g_ref[0]
    a = a_ref[0]
    d = g - a[:, None, :]
    s1_ref[0, 0, 0] = jnp.sum(d, axis=(0, 1))
    s2_ref[0, 0, 0] = jnp.sum(d * d, axis=(0, 1))


def _run_stats(grouped, anch, B, S, K, CIN, TSS=128):
    nj = S // TSS
    return pl.pallas_call(
        _stat_body,
        grid=(B, nj),
        in_specs=[
            pl.BlockSpec((1, TSS, K, CIN), lambda b, j: (b, j, 0, 0)),
            pl.BlockSpec((1, TSS, CIN), lambda b, j: (b, j, 0)),
        ],
        out_specs=[
            pl.BlockSpec((1, 1, 1, CIN), lambda b, j: (b, j, 0, 0)),
            pl.BlockSpec((1, 1, 1, CIN), lambda b, j: (b, j, 0, 0)),
        ],
        out_shape=[
            jax.ShapeDtypeStruct((B, nj, 1, CIN), jnp.float32),
            jax.ShapeDtypeStruct((B, nj, 1, CIN), jnp.float32),
        ],
    )(grouped, anch)


def _conv_body(g_ref, a_ref, s1_ref, s2_ref, al_ref, be_ref, wt_ref, cb_ref,
               y_ref, p1_ref, p2_ref, *, M1, K, CIN, COUT, TS3):
    g = g_ref[0]
    a = a_ref[0]
    s1 = jnp.sum(s1_ref[0])
    s2 = jnp.sum(s2_ref[0])
    var = (s2 - s1 * s1 / M1) / (M1 - 1)
    inv = 1.0 / (jnp.sqrt(var) + 1e-5)
    alpha = al_ref[...]
    beta = be_ref[...]
    xg = (g - a[:, None, :]) * inv * alpha[None] + beta[None]
    rep = jnp.broadcast_to(a[:, None, :], g.shape)
    xc = jnp.concatenate([xg, rep], axis=2).reshape(TS3 * K, 2 * CIN)
    y = lax.dot_general(xc, wt_ref[...], (((1,), (0,)), ((), ())),
                        preferred_element_type=jnp.float32) + cb_ref[...]
    y_ref[0] = y.reshape(TS3, K, COUT)
    p1_ref[0, 0, 0] = jnp.sum(y, axis=0)
    p2_ref[0, 0, 0] = jnp.sum(y * y, axis=0)


def _run_conv(grouped, anch, s1p, s2p, alpha, beta, wt, cb,
              B, S, K, CIN, COUT, TS3=64):
    nj = S // TS3
    nstat = s1p.shape[1]
    M1 = S * K * CIN
    return pl.pallas_call(
        functools.partial(_conv_body, M1=M1, K=K, CIN=CIN, COUT=COUT, TS3=TS3),
        grid=(B, nj),
        in_specs=[
            pl.BlockSpec((1, TS3, K, CIN), lambda b, j: (b, j, 0, 0)),
            pl.BlockSpec((1, TS3, CIN), lambda b, j: (b, j, 0)),
            pl.BlockSpec((1, nstat, CIN), lambda b, j: (b, 0, 0)),
            pl.BlockSpec((1, nstat, CIN), lambda b, j: (b, 0, 0)),
            pl.BlockSpec((1, CIN), lambda b, j: (0, 0)),
            pl.BlockSpec((1, CIN), lambda b, j: (0, 0)),
            pl.BlockSpec((2 * CIN, COUT), lambda b, j: (0, 0)),
            pl.BlockSpec((1, COUT), lambda b, j: (0, 0)),
        ],
        out_specs=[
            pl.BlockSpec((1, TS3, K, COUT), lambda b, j: (b, j, 0, 0)),
            pl.BlockSpec((1, 1, 1, COUT), lambda b, j: (b, j, 0, 0)),
            pl.BlockSpec((1, 1, 1, COUT), lambda b, j: (b, j, 0, 0)),
        ],
        out_shape=[
            jax.ShapeDtypeStruct((B, S, K, COUT), jnp.float32),
            jax.ShapeDtypeStruct((B, nj, 1, COUT), jnp.float32),
            jax.ShapeDtypeStruct((B, nj, 1, COUT), jnp.float32),
        ],
    )(grouped, anch, s1p, s2p, alpha, beta, wt, cb)


def _pool_body(y_ref, p1_ref, p2_ref, ga_ref, bb_ref, o_ref, *, Mg):
    tot1 = jnp.sum(p1_ref[...], axis=0, keepdims=True)
    tot2 = jnp.sum(p2_ref[...], axis=0, keepdims=True)
    mu = tot1 / Mg
    var = tot2 / Mg - mu * mu
    scale = ga_ref[...] * lax.rsqrt(var + 1e-5)
    shift = bb_ref[...] - mu * scale
    y = y_ref[0]
    z = jnp.maximum(y * scale[None] + shift[None], 0.0)
    o_ref[0] = jnp.max(z, axis=1)


def _run_pool(y, p1, p2, gamma, bnb, B, S, K, COUT, TS3=64):
    nj = S // TS3
    Mg = B * S * K
    p1f = p1.reshape(B * nj, COUT)
    p2f = p2.reshape(B * nj, COUT)
    return pl.pallas_call(
        functools.partial(_pool_body, Mg=Mg),
        grid=(B, nj),
        in_specs=[
            pl.BlockSpec((1, TS3, K, COUT), lambda b, j: (b, j, 0, 0)),
            pl.BlockSpec((B * nj, COUT), lambda b, j: (0, 0)),
            pl.BlockSpec((B * nj, COUT), lambda b, j: (0, 0)),
            pl.BlockSpec((1, COUT), lambda b, j: (0, 0)),
            pl.BlockSpec((1, COUT), lambda b, j: (0, 0)),
        ],
        out_specs=pl.BlockSpec((1, TS3, COUT), lambda b, j: (b, j, 0)),
        out_shape=jax.ShapeDtypeStruct((B, S, COUT), jnp.float32),
    )(y, p1f, p2f, gamma, bnb)


def kernel(xyz, points, feature_camera, affine_alpha, affine_beta,
           conv_w, conv_b, bn_gamma, bn_beta):
    B, N, _ = xyz.shape
    CIN = points.shape[2]
    COUT = conv_w.shape[0]
    S, K = 1024, 32

    xyzt = jnp.transpose(xyz, (0, 2, 1))
    pflat = points.reshape(B * N, CIN)
    xc16 = jnp.concatenate(
        [xyz, feature_camera, jnp.zeros((B, N, 122), jnp.float32)], axis=-1
    ).reshape(B * N, 128)

    fps_idx = _run_fps(xyzt, B, N, S).reshape(B * S)

    anch = _sc_gather(pflat, fps_idx)
    xcr = _sc_gather(xc16, fps_idx)
    new_xyz = xcr[:, 0:3].reshape(B, S, 3)
    new_camera = xcr[:, 3:6].reshape(B, S, 3)

    gidx = _run_knn(new_xyz, xyzt, B, N, S, K)
    grouped = _sc_gather(pflat, gidx.reshape(B * S * K)).reshape(B, S, K, CIN)

    anch3 = anch.reshape(B, S, CIN)
    s1p, s2p = _run_stats(grouped, anch3, B, S, K, CIN)
    s1p = s1p.reshape(B, -1, CIN)
    s2p = s2p.reshape(B, -1, CIN)

    alpha = affine_alpha.reshape(1, CIN)
    beta = affine_beta.reshape(1, CIN)
    wt = conv_w.T
    cb = conv_b.reshape(1, COUT)
    y, p1, p2 = _run_conv(grouped, anch3, s1p, s2p, alpha, beta, wt, cb,
                          B, S, K, CIN, COUT)

    out = _run_pool(y, p1, p2, bn_gamma.reshape(1, COUT),
                    bn_beta.reshape(1, COUT), B, S, K, COUT)
    return (new_xyz, out, new_camera)


# speedup vs baseline: 4.8234x; 1.0562x over previous
import functools

import jax
import jax.numpy as jnp
from jax import lax
from jax.experimental import pallas as pl
from jax.experimental.pallas import tpu as pltpu
from jax.experimental.pallas import tpu_sc as plsc


def _fps_body(xyzt_ref, idx_ref, *, N, S):
    x3 = xyzt_ref[0]
    ns, nl = x3.shape[1], x3.shape[2]
    ss, sl = idx_ref.shape[1], idx_ref.shape[2]
    iota_n = (lax.broadcasted_iota(jnp.int32, (ns, nl), 0) * nl
              + lax.broadcasted_iota(jnp.int32, (ns, nl), 1))
    iota_s = (lax.broadcasted_iota(jnp.int32, (ss, sl), 0) * sl
              + lax.broadcasted_iota(jnp.int32, (ss, sl), 1))

    def body(i, carry):
        dists, far, idxv = carry
        idxv = jnp.where(iota_s == i, far, idxv)
        oh = iota_n == far
        c = jnp.sum(jnp.where(oh[None], x3, 0.0), axis=(1, 2),
                    keepdims=True)
        d = jnp.sum((x3 - c) ** 2, axis=0)
        dists = jnp.minimum(dists, d)
        m = jnp.max(dists, axis=(0, 1), keepdims=True)
        far = jnp.min(jnp.where(dists == m, iota_n, N), axis=(0, 1),
                      keepdims=True)
        return dists, far, idxv

    dists0 = jnp.full((ns, nl), 1e10, jnp.float32)
    far0 = jnp.zeros((1, 1), jnp.int32)
    idxv0 = jnp.zeros((ss, sl), jnp.int32)
    _, _, idxv = lax.fori_loop(0, S, body, (dists0, far0, idxv0))
    idx_ref[...] = (idxv + pl.program_id(0) * N)[None]


def _run_fps(xyzt, B, N, S):
    xyzr = xyzt.reshape(B, 3, 8, N // 8)
    return pl.pallas_call(
        functools.partial(_fps_body, N=N, S=S),
        grid=(B,),
        in_specs=[pl.BlockSpec((1, 3, 8, N // 8), lambda b: (b, 0, 0, 0))],
        out_specs=pl.BlockSpec((1, 8, S // 8), lambda b: (b, 0, 0)),
        out_shape=jax.ShapeDtypeStruct((B, 8, S // 8), jnp.int32),
    )(xyzr)


def _sc_gather(table, idx, chunk=128):
    rows, depth = idx.shape[0], table.shape[1]
    info = plsc.get_sparse_core_info()
    ncores = info.num_cores
    nworkers = ncores * info.num_subcores
    per_w = rows // nworkers
    n_chunks = per_w // chunk
    mesh = plsc.VectorSubcoreMesh(core_axis_name="c", subcore_axis_name="s")

    @functools.partial(
        pl.kernel,
        mesh=mesh,
        out_type=jax.ShapeDtypeStruct((rows, depth), jnp.float32),
        scratch_types=[
            pltpu.VMEM((chunk,), jnp.int32),
            pltpu.VMEM((chunk, depth), jnp.float32),
            pltpu.SemaphoreType.DMA,
        ],
    )
    def gk(table_hbm, idx_hbm, out_hbm, idx_v, rows_v, sem):
        wid = lax.axis_index("s") * ncores + lax.axis_index("c")
        base = wid * per_w

        def body(i, carry):
            off = base + i * chunk
            pltpu.sync_copy(idx_hbm.at[pl.ds(off, chunk)], idx_v)
            pltpu.async_copy(table_hbm.at[idx_v], rows_v, sem).wait()
            pltpu.sync_copy(rows_v, out_hbm.at[pl.ds(off, chunk)])
            return carry

        lax.fori_loop(0, n_chunks, body, 0)

    return gk(table, idx)


def _knn_body(q_ref, kt_ref, idx_ref, *, N, K, TS):
    q = q_ref[0]
    kt = kt_ref[0]
    qk = lax.dot_general(q, kt, (((1,), (0,)), ((), ())),
                         preferred_element_type=jnp.float32)
    q2 = jnp.sum(q * q, axis=1, keepdims=True)
    k2 = jnp.sum(kt * kt, axis=0, keepdims=True)
    dmat = q2 - 2.0 * qk + k2
    iota = lax.broadcasted_iota(jnp.int32, (TS, N), 1)
    iota_k = lax.broadcasted_iota(jnp.int32, (TS, K), 1)
    acc = jnp.zeros((TS, K), jnp.int32)
    for k in range(K):
        a = jnp.argmin(dmat, axis=1).astype(jnp.int32)[:, None]
        acc = jnp.where(iota_k == k, a, acc)
        dmat = jnp.where(iota == a, 1e30, dmat)
    idx_ref[0] = acc + pl.program_id(0) * N


def _run_knn(new_xyz, xyzt, B, N, S, K, TS=128):
    return pl.pallas_call(
        functools.partial(_knn_body, N=N, K=K, TS=TS),
        grid=(B, S // TS),
        in_specs=[
            pl.BlockSpec((1, TS, 3), lambda b, j: (b, j, 0)),
            pl.BlockSpec((1, 3, N), lambda b, j: (b, 0, 0)),
        ],
        out_specs=pl.BlockSpec((1, TS, K), lambda b, j: (b, j, 0)),
        out_shape=jax.ShapeDtypeStruct((B, S, K), jnp.int32),
    )(new_xyz, xyzt)


def _stat_body(g_ref, a_ref, s1_ref, s2_ref):
    g = g_ref[0]
    a = a_ref[0]
    d = g - a[:, None, :]
    s1_ref[0, 0, 0] = jnp.sum(d, axis=(0, 1))
    s2_ref[0, 0, 0] = jnp.sum(d * d, axis=(0, 1))


def _run_stats(grouped, anch, B, S, K, CIN, TSS=128):
    nj = S // TSS
    return pl.pallas_call(
        _stat_body,
        grid=(B, nj),
        in_specs=[
            pl.BlockSpec((1, TSS, K, CIN), lambda b, j: (b, j, 0, 0)),
            pl.BlockSpec((1, TSS, CIN), lambda b, j: (b, j, 0)),
        ],
        out_specs=[
            pl.BlockSpec((1, 1, 1, CIN), lambda b, j: (b, j, 0, 0)),
            pl.BlockSpec((1, 1, 1, CIN), lambda b, j: (b, j, 0, 0)),
        ],
        out_shape=[
            jax.ShapeDtypeStruct((B, nj, 1, CIN), jnp.float32),
            jax.ShapeDtypeStruct((B, nj, 1, CIN), jnp.float32),
        ],
    )(grouped, anch)


def _conv_body(g_ref, a_ref, s1_ref, s2_ref, al_ref, be_ref, wt_ref, cb_ref,
               y_ref, p1_ref, p2_ref, *, M1, K, CIN, COUT, TS3):
    g = g_ref[0]
    a = a_ref[0]
    s1 = jnp.sum(s1_ref[0])
    s2 = jnp.sum(s2_ref[0])
    var = (s2 - s1 * s1 / M1) / (M1 - 1)
    inv = 1.0 / (jnp.sqrt(var) + 1e-5)
    alpha = al_ref[...]
    beta = be_ref[...]
    xg = (g - a[:, None, :]) * inv * alpha[None] + beta[None]
    rep = jnp.broadcast_to(a[:, None, :], g.shape)
    xc = jnp.concatenate([xg, rep], axis=2).reshape(TS3 * K, 2 * CIN)
    y = lax.dot_general(xc, wt_ref[...], (((1,), (0,)), ((), ())),
                        preferred_element_type=jnp.float32) + cb_ref[...]
    y_ref[0] = y.reshape(TS3, K, COUT)
    p1_ref[0, 0, 0] = jnp.sum(y, axis=0)
    p2_ref[0, 0, 0] = jnp.sum(y * y, axis=0)


def _run_conv(grouped, anch, s1p, s2p, alpha, beta, wt, cb,
              B, S, K, CIN, COUT, TS3=64):
    nj = S // TS3
    nstat = s1p.shape[1]
    M1 = S * K * CIN
    return pl.pallas_call(
        functools.partial(_conv_body, M1=M1, K=K, CIN=CIN, COUT=COUT, TS3=TS3),
        grid=(B, nj),
        in_specs=[
            pl.BlockSpec((1, TS3, K, CIN), lambda b, j: (b, j, 0, 0)),
            pl.BlockSpec((1, TS3, CIN), lambda b, j: (b, j, 0)),
            pl.BlockSpec((1, nstat, CIN), lambda b, j: (b, 0, 0)),
            pl.BlockSpec((1, nstat, CIN), lambda b, j: (b, 0, 0)),
            pl.BlockSpec((1, CIN), lambda b, j: (0, 0)),
            pl.BlockSpec((1, CIN), lambda b, j: (0, 0)),
            pl.BlockSpec((2 * CIN, COUT), lambda b, j: (0, 0)),
            pl.BlockSpec((1, COUT), lambda b, j: (0, 0)),
        ],
        out_specs=[
            pl.BlockSpec((1, TS3, K, COUT), lambda b, j: (b, j, 0, 0)),
            pl.BlockSpec((1, 1, 1, COUT), lambda b, j: (b, j, 0, 0)),
            pl.BlockSpec((1, 1, 1, COUT), lambda b, j: (b, j, 0, 0)),
        ],
        out_shape=[
            jax.ShapeDtypeStruct((B, S, K, COUT), jnp.float32),
            jax.ShapeDtypeStruct((B, nj, 1, COUT), jnp.float32),
            jax.ShapeDtypeStruct((B, nj, 1, COUT), jnp.float32),
        ],
    )(grouped, anch, s1p, s2p, alpha, beta, wt, cb)


def _pool_body(y_ref, p1_ref, p2_ref, ga_ref, bb_ref, o_ref, *, Mg):
    tot1 = jnp.sum(p1_ref[...], axis=0, keepdims=True)
    tot2 = jnp.sum(p2_ref[...], axis=0, keepdims=True)
    mu = tot1 / Mg
    var = tot2 / Mg - mu * mu
    scale = ga_ref[...] * lax.rsqrt(var + 1e-5)
    shift = bb_ref[...] - mu * scale
    y = y_ref[0]
    z = jnp.maximum(y * scale[None] + shift[None], 0.0)
    o_ref[0] = jnp.max(z, axis=1)


def _run_pool(y, p1, p2, gamma, bnb, B, S, K, COUT, TS3=64):
    nj = S // TS3
    Mg = B * S * K
    p1f = p1.reshape(B * nj, COUT)
    p2f = p2.reshape(B * nj, COUT)
    return pl.pallas_call(
        functools.partial(_pool_body, Mg=Mg),
        grid=(B, nj),
        in_specs=[
            pl.BlockSpec((1, TS3, K, COUT), lambda b, j: (b, j, 0, 0)),
            pl.BlockSpec((B * nj, COUT), lambda b, j: (0, 0)),
            pl.BlockSpec((B * nj, COUT), lambda b, j: (0, 0)),
            pl.BlockSpec((1, COUT), lambda b, j: (0, 0)),
            pl.BlockSpec((1, COUT), lambda b, j: (0, 0)),
        ],
        out_specs=pl.BlockSpec((1, TS3, COUT), lambda b, j: (b, j, 0)),
        out_shape=jax.ShapeDtypeStruct((B, S, COUT), jnp.float32),
    )(y, p1f, p2f, gamma, bnb)


def kernel(xyz, points, feature_camera, affine_alpha, affine_beta,
           conv_w, conv_b, bn_gamma, bn_beta):
    B, N, _ = xyz.shape
    CIN = points.shape[2]
    COUT = conv_w.shape[0]
    S, K = 1024, 32

    xyzt = jnp.transpose(xyz, (0, 2, 1))
    pflat = points.reshape(B * N, CIN)
    xc16 = jnp.concatenate(
        [xyz, feature_camera, jnp.zeros((B, N, 122), jnp.float32)], axis=-1
    ).reshape(B * N, 128)

    fps_idx = _run_fps(xyzt, B, N, S).reshape(B * S)


    anch = _sc_gather(pflat, fps_idx)
    xcr = _sc_gather(xc16, fps_idx)
    new_xyz = xcr[:, 0:3].reshape(B, S, 3)
    new_camera = xcr[:, 3:6].reshape(B, S, 3)

    gidx = _run_knn(new_xyz, xyzt, B, N, S, K)
    grouped = _sc_gather(pflat, gidx.reshape(B * S * K)).reshape(B, S, K, CIN)

    anch3 = anch.reshape(B, S, CIN)
    s1p, s2p = _run_stats(grouped, anch3, B, S, K, CIN)
    s1p = s1p.reshape(B, -1, CIN)
    s2p = s2p.reshape(B, -1, CIN)

    alpha = affine_alpha.reshape(1, CIN)
    beta = affine_beta.reshape(1, CIN)
    wt = conv_w.T
    cb = conv_b.reshape(1, COUT)
    y, p1, p2 = _run_conv(grouped, anch3, s1p, s2p, alpha, beta, wt, cb,
                          B, S, K, CIN, COUT)

    out = _run_pool(y, p1, p2, bn_gamma.reshape(1, COUT),
                    bn_beta.reshape(1, COUT), B, S, K, COUT)
    return (new_xyz, out, new_camera)


# speedup vs baseline: 11.0806x; 2.2973x over previous
import functools

import jax
import jax.numpy as jnp
from jax import lax
from jax.experimental import pallas as pl
from jax.experimental.pallas import tpu as pltpu
from jax.experimental.pallas import tpu_sc as plsc


def _fps_body(xyzt_ref, idx_ref, *, B, N, S):
    x3 = xyzt_ref[...]
    iota_n = lax.broadcasted_iota(jnp.int32, (B, N), 1)
    iota_s = lax.broadcasted_iota(jnp.int32, (B, S), 1)

    def body(i, carry):
        dists, far, idxv = carry
        idxv = jnp.where(iota_s == i, far, idxv)
        oh = iota_n == far
        c = jnp.sum(jnp.where(oh[None], x3, 0.0), axis=2,
                    keepdims=True)
        d = jnp.sum((x3 - c) ** 2, axis=0)
        dists = jnp.minimum(dists, d)
        m = jnp.max(dists, axis=1, keepdims=True)
        far = jnp.min(jnp.where(dists == m, iota_n, N), axis=1,
                      keepdims=True)
        return dists, far, idxv

    z = x3[0] * 0.0
    dists0 = z + 1e10
    idxv0 = z[:, :S].astype(jnp.int32)
    far0 = idxv0[:, :1]
    _, _, idxv = lax.fori_loop(0, S, body, (dists0, far0, idxv0))
    bio = lax.broadcasted_iota(jnp.int32, (B, S), 0) * N
    idx_ref[...] = idxv + bio


def _run_fps(xyzt, B, N, S):
    xyzb = jnp.transpose(xyzt, (1, 0, 2))
    return pl.pallas_call(
        functools.partial(_fps_body, B=B, N=N, S=S),
        in_specs=[pl.BlockSpec((3, B, N), lambda: (0, 0, 0))],
        out_specs=pl.BlockSpec((B, S), lambda: (0, 0)),
        out_shape=jax.ShapeDtypeStruct((B, S), jnp.int32),
    )(xyzb)


def _sc_gather(table, idx, chunk=128):
    rows, depth = idx.shape[0], table.shape[1]
    info = plsc.get_sparse_core_info()
    ncores = info.num_cores
    nworkers = ncores * info.num_subcores
    per_w = rows // nworkers
    n_chunks = per_w // chunk
    mesh = plsc.VectorSubcoreMesh(core_axis_name="c", subcore_axis_name="s")

    @functools.partial(
        pl.kernel,
        mesh=mesh,
        out_type=jax.ShapeDtypeStruct((rows, depth), jnp.float32),
        scratch_types=[
            pltpu.VMEM((chunk,), jnp.int32),
            pltpu.VMEM((chunk, depth), jnp.float32),
            pltpu.SemaphoreType.DMA,
        ],
    )
    def gk(table_hbm, idx_hbm, out_hbm, idx_v, rows_v, sem):
        wid = lax.axis_index("s") * ncores + lax.axis_index("c")
        base = wid * per_w

        def body(i, carry):
            off = base + i * chunk
            pltpu.sync_copy(idx_hbm.at[pl.ds(off, chunk)], idx_v)
            pltpu.async_copy(table_hbm.at[idx_v], rows_v, sem).wait()
            pltpu.sync_copy(rows_v, out_hbm.at[pl.ds(off, chunk)])
            return carry

        lax.fori_loop(0, n_chunks, body, 0)

    return gk(table, idx)


def _knn_body(q_ref, kt_ref, idx_ref, *, N, K, TS):
    q = q_ref[0]
    kt = kt_ref[0]
    qk = lax.dot_general(q, kt, (((1,), (0,)), ((), ())),
                         preferred_element_type=jnp.float32)
    q2 = jnp.sum(q * q, axis=1, keepdims=True)
    k2 = jnp.sum(kt * kt, axis=0, keepdims=True)
    dmat = q2 - 2.0 * qk + k2
    iota = lax.broadcasted_iota(jnp.int32, (TS, N), 1)
    iota_k = lax.broadcasted_iota(jnp.int32, (TS, K), 1)
    acc = jnp.zeros((TS, K), jnp.int32)
    for k in range(K):
        a = jnp.argmin(dmat, axis=1).astype(jnp.int32)[:, None]
        acc = jnp.where(iota_k == k, a, acc)
        dmat = jnp.where(iota == a, 1e30, dmat)
    idx_ref[0] = acc + pl.program_id(0) * N


def _run_knn(new_xyz, xyzt, B, N, S, K, TS=128):
    return pl.pallas_call(
        functools.partial(_knn_body, N=N, K=K, TS=TS),
        grid=(B, S // TS),
        in_specs=[
            pl.BlockSpec((1, TS, 3), lambda b, j: (b, j, 0)),
            pl.BlockSpec((1, 3, N), lambda b, j: (b, 0, 0)),
        ],
        out_specs=pl.BlockSpec((1, TS, K), lambda b, j: (b, j, 0)),
        out_shape=jax.ShapeDtypeStruct((B, S, K), jnp.int32),
    )(new_xyz, xyzt)


def _stat_body(g_ref, a_ref, s1_ref, s2_ref):
    g = g_ref[0]
    a = a_ref[0]
    d = g - a[:, None, :]
    s1_ref[0, 0, 0] = jnp.sum(d, axis=(0, 1))
    s2_ref[0, 0, 0] = jnp.sum(d * d, axis=(0, 1))


def _run_stats(grouped, anch, B, S, K, CIN, TSS=128):
    nj = S // TSS
    return pl.pallas_call(
        _stat_body,
        grid=(B, nj),
        in_specs=[
            pl.BlockSpec((1, TSS, K, CIN), lambda b, j: (b, j, 0, 0)),
            pl.BlockSpec((1, TSS, CIN), lambda b, j: (b, j, 0)),
        ],
        out_specs=[
            pl.BlockSpec((1, 1, 1, CIN), lambda b, j: (b, j, 0, 0)),
            pl.BlockSpec((1, 1, 1, CIN), lambda b, j: (b, j, 0, 0)),
        ],
        out_shape=[
            jax.ShapeDtypeStruct((B, nj, 1, CIN), jnp.float32),
            jax.ShapeDtypeStruct((B, nj, 1, CIN), jnp.float32),
        ],
    )(grouped, anch)


def _conv_body(g_ref, a_ref, s1_ref, s2_ref, al_ref, be_ref, wt_ref, cb_ref,
               y_ref, p1_ref, p2_ref, *, M1, K, CIN, COUT, TS3):
    g = g_ref[0]
    a = a_ref[0]
    s1 = jnp.sum(s1_ref[0])
    s2 = jnp.sum(s2_ref[0])
    var = (s2 - s1 * s1 / M1) / (M1 - 1)
    inv = 1.0 / (jnp.sqrt(var) + 1e-5)
    alpha = al_ref[...]
    beta = be_ref[...]
    xg = (g - a[:, None, :]) * inv * alpha[None] + beta[None]
    rep = jnp.broadcast_to(a[:, None, :], g.shape)
    xc = jnp.concatenate([xg, rep], axis=2).reshape(TS3 * K, 2 * CIN)
    y = lax.dot_general(xc, wt_ref[...], (((1,), (0,)), ((), ())),
                        preferred_element_type=jnp.float32) + cb_ref[...]
    y_ref[0] = y.reshape(TS3, K, COUT)
    p1_ref[0, 0, 0] = jnp.sum(y, axis=0)
    p2_ref[0, 0, 0] = jnp.sum(y * y, axis=0)


def _run_conv(grouped, anch, s1p, s2p, alpha, beta, wt, cb,
              B, S, K, CIN, COUT, TS3=64):
    nj = S // TS3
    nstat = s1p.shape[1]
    M1 = S * K * CIN
    return pl.pallas_call(
        functools.partial(_conv_body, M1=M1, K=K, CIN=CIN, COUT=COUT, TS3=TS3),
        grid=(B, nj),
        in_specs=[
            pl.BlockSpec((1, TS3, K, CIN), lambda b, j: (b, j, 0, 0)),
            pl.BlockSpec((1, TS3, CIN), lambda b, j: (b, j, 0)),
            pl.BlockSpec((1, nstat, CIN), lambda b, j: (b, 0, 0)),
            pl.BlockSpec((1, nstat, CIN), lambda b, j: (b, 0, 0)),
            pl.BlockSpec((1, CIN), lambda b, j: (0, 0)),
            pl.BlockSpec((1, CIN), lambda b, j: (0, 0)),
            pl.BlockSpec((2 * CIN, COUT), lambda b, j: (0, 0)),
            pl.BlockSpec((1, COUT), lambda b, j: (0, 0)),
        ],
        out_specs=[
            pl.BlockSpec((1, TS3, K, COUT), lambda b, j: (b, j, 0, 0)),
            pl.BlockSpec((1, 1, 1, COUT), lambda b, j: (b, j, 0, 0)),
            pl.BlockSpec((1, 1, 1, COUT), lambda b, j: (b, j, 0, 0)),
        ],
        out_shape=[
            jax.ShapeDtypeStruct((B, S, K, COUT), jnp.float32),
            jax.ShapeDtypeStruct((B, nj, 1, COUT), jnp.float32),
            jax.ShapeDtypeStruct((B, nj, 1, COUT), jnp.float32),
        ],
    )(grouped, anch, s1p, s2p, alpha, beta, wt, cb)


def _pool_body(y_ref, p1_ref, p2_ref, ga_ref, bb_ref, o_ref, *, Mg):
    tot1 = jnp.sum(p1_ref[...], axis=0, keepdims=True)
    tot2 = jnp.sum(p2_ref[...], axis=0, keepdims=True)
    mu = tot1 / Mg
    var = tot2 / Mg - mu * mu
    scale = ga_ref[...] * lax.rsqrt(var + 1e-5)
    shift = bb_ref[...] - mu * scale
    y = y_ref[0]
    z = jnp.maximum(y * scale[None] + shift[None], 0.0)
    o_ref[0] = jnp.max(z, axis=1)


def _run_pool(y, p1, p2, gamma, bnb, B, S, K, COUT, TS3=64):
    nj = S // TS3
    Mg = B * S * K
    p1f = p1.reshape(B * nj, COUT)
    p2f = p2.reshape(B * nj, COUT)
    return pl.pallas_call(
        functools.partial(_pool_body, Mg=Mg),
        grid=(B, nj),
        in_specs=[
            pl.BlockSpec((1, TS3, K, COUT), lambda b, j: (b, j, 0, 0)),
            pl.BlockSpec((B * nj, COUT), lambda b, j: (0, 0)),
            pl.BlockSpec((B * nj, COUT), lambda b, j: (0, 0)),
            pl.BlockSpec((1, COUT), lambda b, j: (0, 0)),
            pl.BlockSpec((1, COUT), lambda b, j: (0, 0)),
        ],
        out_specs=pl.BlockSpec((1, TS3, COUT), lambda b, j: (b, j, 0)),
        out_shape=jax.ShapeDtypeStruct((B, S, COUT), jnp.float32),
    )(y, p1f, p2f, gamma, bnb)


def kernel(xyz, points, feature_camera, affine_alpha, affine_beta,
           conv_w, conv_b, bn_gamma, bn_beta):
    B, N, _ = xyz.shape
    CIN = points.shape[2]
    COUT = conv_w.shape[0]
    S, K = 1024, 32

    xyzt = jnp.transpose(xyz, (0, 2, 1))
    pflat = points.reshape(B * N, CIN)
    xc16 = jnp.concatenate(
        [xyz, feature_camera, jnp.zeros((B, N, 122), jnp.float32)], axis=-1
    ).reshape(B * N, 128)

    fps_idx = _run_fps(xyzt, B, N, S).reshape(B * S)


    anch = _sc_gather(pflat, fps_idx)
    xcr = _sc_gather(xc16, fps_idx)
    new_xyz = xcr[:, 0:3].reshape(B, S, 3)
    new_camera = xcr[:, 3:6].reshape(B, S, 3)

    gidx = _run_knn(new_xyz, xyzt, B, N, S, K)
    grouped = _sc_gather(pflat, gidx.reshape(B * S * K)).reshape(B, S, K, CIN)

    anch3 = anch.reshape(B, S, CIN)
    s1p, s2p = _run_stats(grouped, anch3, B, S, K, CIN)
    s1p = s1p.reshape(B, -1, CIN)
    s2p = s2p.reshape(B, -1, CIN)

    alpha = affine_alpha.reshape(1, CIN)
    beta = affine_beta.reshape(1, CIN)
    wt = conv_w.T
    cb = conv_b.reshape(1, COUT)
    y, p1, p2 = _run_conv(grouped, anch3, s1p, s2p, alpha, beta, wt, cb,
                          B, S, K, CIN, COUT)

    out = _run_pool(y, p1, p2, bn_gamma.reshape(1, COUT),
                    bn_beta.reshape(1, COUT), B, S, K, COUT)
    return (new_xyz, out, new_camera)
